# baseline (device time: 21063 ns/iter reference)
import jax
import jax.numpy as jnp
from jax import lax
from jax.experimental import pallas as pl
from jax.experimental.pallas import tpu as pltpu

N_DEV = 8
E_TOTAL = 32
BLK = 64


def kernel(x, router_W, route_idx, expert_W):
    n_tok, d_model = x.shape
    e_local, _, d_ff = expert_W.shape

    def body(x_ref, rw_ref, idx_ref, ew_ref, out_ref,
             xcat_ref, stage_ref, rs_ref,
             s1_sems, p1_sems, s2_sems, p2_sems):
        my_pos = lax.axis_index("i")

        barrier_sem = pltpu.get_barrier_semaphore()
        for k in range(1, N_DEV):
            pl.semaphore_signal(
                barrier_sem, inc=1,
                device_id=(lax.rem(my_pos + k, N_DEV),),
                device_id_type=pl.DeviceIdType.MESH,
            )

        xf = x_ref[:, :]
        scores = jnp.dot(xf, rw_ref[:, :], preferred_element_type=jnp.float32)
        m = jnp.max(scores, axis=-1, keepdims=True)
        p = jnp.exp(scores - m)
        p = p / jnp.sum(p, axis=-1, keepdims=True)

        cols = lax.broadcasted_iota(jnp.int32, (n_tok, E_TOTAL), 1)
        oh0 = (cols == idx_ref[:, 0:1]).astype(jnp.float32)
        oh1 = (cols == idx_ref[:, 1:2]).astype(jnp.float32)
        g0 = jnp.sum(p * oh0, axis=-1, keepdims=True)
        g1 = jnp.sum(p * oh1, axis=-1, keepdims=True)
        gs = g0 + g1
        gate = (g0 / gs) * oh0 + (g1 / gs) * oh1

        base = my_pos * e_local
        xw = []
        for e in range(e_local):
            sel = (cols == base + e).astype(jnp.float32)
            w = jnp.sum(gate * sel, axis=-1, keepdims=True)
            xw.append(xf * w)
        xcat_ref[:, :] = jnp.concatenate(xw, axis=1).astype(jnp.bfloat16)
        wcat = ew_ref[:, :, :].reshape(e_local * d_model, d_ff).astype(jnp.bfloat16)

        pl.semaphore_wait(barrier_sem, N_DEV - 1)

        sends1 = []
        for k in range(1, N_DEV):
            t = lax.rem(my_pos + k, N_DEV)
            xb = xcat_ref[pl.ds(t * BLK, BLK), :]
            pb = jnp.dot(xb, wcat, preferred_element_type=jnp.float32)
            j = N_DEV - k
            stage_ref[j, :, :] = pb.astype(jnp.bfloat16)
            rdma = pltpu.make_async_remote_copy(
                src_ref=stage_ref.at[j],
                dst_ref=rs_ref.at[j],
                send_sem=s1_sems.at[j],
                recv_sem=p1_sems.at[j],
                device_id=(t,),
                device_id_type=pl.DeviceIdType.MESH,
            )
            rdma.start()
            sends1.append(rdma)

        xb = xcat_ref[pl.ds(my_pos * BLK, BLK), :]
        reduced = jnp.dot(xb, wcat, preferred_element_type=jnp.float32)

        for j in range(1, N_DEV):
            recv = pltpu.make_async_remote_copy(
                src_ref=stage_ref.at[j], dst_ref=rs_ref.at[j],
                send_sem=s1_sems.at[j], recv_sem=p1_sems.at[j],
                device_id=(my_pos,), device_id_type=pl.DeviceIdType.MESH,
            )
            recv.wait_recv()
        for j in range(1, N_DEV):
            reduced = reduced + rs_ref[j, :, :].astype(jnp.float32)

        out_ref[pl.ds(my_pos * BLK, BLK), :] = reduced.astype(jnp.bfloat16)
        sends2 = []
        for k in range(1, N_DEV):
            t = lax.rem(my_pos + k, N_DEV)
            j = N_DEV - k
            rdma = pltpu.make_async_remote_copy(
                src_ref=out_ref.at[pl.ds(my_pos * BLK, BLK)],
                dst_ref=out_ref.at[pl.ds(my_pos * BLK, BLK)],
                send_sem=s2_sems.at[j],
                recv_sem=p2_sems.at[j],
                device_id=(t,),
                device_id_type=pl.DeviceIdType.MESH,
            )
            rdma.start()
            sends2.append(rdma)

        for rdma in sends1:
            rdma.wait_send()

        for j in range(1, N_DEV):
            s = lax.rem(my_pos + N_DEV - j, N_DEV)
            recv = pltpu.make_async_remote_copy(
                src_ref=out_ref.at[pl.ds(s * BLK, BLK)],
                dst_ref=out_ref.at[pl.ds(s * BLK, BLK)],
                send_sem=s2_sems.at[j], recv_sem=p2_sems.at[j],
                device_id=(my_pos,), device_id_type=pl.DeviceIdType.MESH,
            )
            recv.wait_recv()
        for rdma in sends2:
            rdma.wait_send()

    return pl.pallas_call(
        body,
        out_shape=jax.ShapeDtypeStruct((n_tok, d_ff), jnp.bfloat16),
        in_specs=[pl.BlockSpec(memory_space=pltpu.VMEM)] * 4,
        out_specs=pl.BlockSpec(memory_space=pltpu.VMEM),
        scratch_shapes=[
            pltpu.VMEM((n_tok, e_local * d_model), jnp.bfloat16),
            pltpu.VMEM((N_DEV, BLK, d_ff), jnp.bfloat16),
            pltpu.VMEM((N_DEV, BLK, d_ff), jnp.bfloat16),
            pltpu.SemaphoreType.DMA((N_DEV,)),
            pltpu.SemaphoreType.DMA((N_DEV,)),
            pltpu.SemaphoreType.DMA((N_DEV,)),
            pltpu.SemaphoreType.DMA((N_DEV,)),
        ],
        compiler_params=pltpu.CompilerParams(collective_id=0),
    )(x, router_W, route_idx, expert_W)
